# baseline (speedup 1.0000x reference)
import jax
import jax.numpy as jnp
from jax import lax
from jax.experimental import pallas as pl
from jax.experimental.pallas import tpu as pltpu
from jax.experimental.pallas import tpu_sc as plsc

_B, _L = 16, 2048
_LANES = 16
_CHUNKS = _L // _LANES


def _split_roll_body(ids_hbm, tt_hbm, out1_hbm, out2_hbm,
                     ids_v, tt_v, out1_v, in2_v, out2_v, acc_v, sem1, sem2):
    c = lax.axis_index("c")
    s = lax.axis_index("s")
    wid = s

    if True:
        cpa = pltpu.make_async_copy(ids_hbm.at[wid], ids_v, sem1)
        cpb = pltpu.make_async_copy(tt_hbm.at[wid], tt_v, sem2)
        cpa.start()
        cpb.start()
        cpa.wait()
        cpb.wait()
        acc_v[...] = jnp.zeros((_LANES,), jnp.int32)

        def pass1(j, carry):
            base = j * _LANES
            ids = ids_v[pl.ds(base, _LANES)]
            tt = tt_v[pl.ds(base, _LANES)]
            o1 = ids * (1 - tt)
            out1_v[pl.ds(base, _LANES)] = o1
            i2 = ids - o1
            in2_v[pl.ds(base, _LANES)] = i2
            in2_v[pl.ds(base + _L, _LANES)] = i2
            acc_v[...] = acc_v[...] + jnp.minimum(o1, 1)
            return carry

        lax.fori_loop(0, _CHUNKS, pass1, 0, unroll=2)
        accv = acc_v[...]
        n = accv[0]
        for lane in range(1, _LANES):
            n = n + accv[lane]

        cp1 = pltpu.make_async_copy(out1_v, out1_hbm.at[wid], sem1)
        cp1.start()

        def pass2(j, carry):
            base = j * _LANES
            out2_v[pl.ds(base, _LANES)] = in2_v[pl.ds(base + n, _LANES)]
            return carry

        lax.fori_loop(0, _CHUNKS, pass2, 0, unroll=2)
        pltpu.sync_copy(out2_v, out2_hbm.at[wid])
        cp1.wait()


def kernel(l_input_ids, token_type_ids):
    mesh = plsc.VectorSubcoreMesh(core_axis_name="c", subcore_axis_name="s",
                                  num_cores=1)
    f = pl.kernel(
        _split_roll_body,
        mesh=mesh,
        out_type=(
            jax.ShapeDtypeStruct((_B, _L), jnp.int32),
            jax.ShapeDtypeStruct((_B, _L), jnp.int32),
        ),
        scratch_types=[
            pltpu.VMEM((_L,), jnp.int32),
            pltpu.VMEM((_L,), jnp.int32),
            pltpu.VMEM((_L,), jnp.int32),
            pltpu.VMEM((2 * _L,), jnp.int32),
            pltpu.VMEM((_L,), jnp.int32),
            pltpu.VMEM((_LANES,), jnp.int32),
            pltpu.SemaphoreType.DMA,
            pltpu.SemaphoreType.DMA,
        ],
    )
    return f(l_input_ids, token_type_ids)


# speedup vs baseline: 1.0104x; 1.0104x over previous
import jax
import jax.numpy as jnp
from jax import lax
from jax.experimental import pallas as pl
from jax.experimental.pallas import tpu as pltpu
from jax.experimental.pallas import tpu_sc as plsc

_B, _L = 16, 2048
_LANES = 16
_CHUNKS = _L // _LANES


def _split_roll_body(ids_hbm, tt_hbm, out1_hbm, out2_hbm,
                     ids_v, tt_v, out1_v, in2_v, out2_v, acc_v, sem1, sem2):
    c = lax.axis_index("c")
    s = lax.axis_index("s")
    wid = s

    if True:
        cpa = pltpu.make_async_copy(ids_hbm.at[wid], ids_v, sem1)
        cpb = pltpu.make_async_copy(tt_hbm.at[wid], tt_v, sem2)
        cpa.start()
        cpb.start()
        cpa.wait()
        cpb.wait()
        acc_v[...] = jnp.zeros((_LANES,), jnp.int32)

        def pass1(j, carry):
            base = j * _LANES
            ids = ids_v[pl.ds(base, _LANES)]
            tt = tt_v[pl.ds(base, _LANES)]
            o1 = ids * (1 - tt)
            out1_v[pl.ds(base, _LANES)] = o1
            i2 = ids - o1
            in2_v[pl.ds(base, _LANES)] = i2
            in2_v[pl.ds(base + _L, _LANES)] = i2
            acc_v[...] = acc_v[...] + jnp.minimum(o1, 1)
            return carry

        lax.fori_loop(0, _CHUNKS, pass1, 0, unroll=2)
        accv = acc_v[...]
        n = accv[0]
        for lane in range(1, _LANES):
            n = n + accv[lane]

        cp1 = pltpu.make_async_copy(out1_v, out1_hbm.at[wid], sem1)
        cp1.start()

        def pass2(j, carry):
            base = j * _LANES
            out2_v[pl.ds(base, _LANES)] = in2_v[pl.ds(base + n, _LANES)]
            return carry

        half = _L // 2
        lax.fori_loop(0, _CHUNKS // 2, pass2, 0, unroll=2)
        cp2a = pltpu.make_async_copy(out2_v.at[pl.ds(0, half)],
                                     out2_hbm.at[wid, pl.ds(0, half)], sem2)
        cp2a.start()
        lax.fori_loop(_CHUNKS // 2, _CHUNKS, pass2, 0, unroll=2)
        cp2b = pltpu.make_async_copy(out2_v.at[pl.ds(half, half)],
                                     out2_hbm.at[wid, pl.ds(half, half)], sem2)
        cp2b.start()
        cp2a.wait()
        cp2b.wait()
        cp1.wait()


def kernel(l_input_ids, token_type_ids):
    mesh = plsc.VectorSubcoreMesh(core_axis_name="c", subcore_axis_name="s",
                                  num_cores=1)
    f = pl.kernel(
        _split_roll_body,
        mesh=mesh,
        out_type=(
            jax.ShapeDtypeStruct((_B, _L), jnp.int32),
            jax.ShapeDtypeStruct((_B, _L), jnp.int32),
        ),
        scratch_types=[
            pltpu.VMEM((_L,), jnp.int32),
            pltpu.VMEM((_L,), jnp.int32),
            pltpu.VMEM((_L,), jnp.int32),
            pltpu.VMEM((2 * _L,), jnp.int32),
            pltpu.VMEM((_L,), jnp.int32),
            pltpu.VMEM((_LANES,), jnp.int32),
            pltpu.SemaphoreType.DMA,
            pltpu.SemaphoreType.DMA,
        ],
    )
    return f(l_input_ids, token_type_ids)


# speedup vs baseline: 1.0143x; 1.0039x over previous
import jax
import jax.numpy as jnp
from jax import lax
from jax.experimental import pallas as pl
from jax.experimental.pallas import tpu as pltpu
from jax.experimental.pallas import tpu_sc as plsc

_B, _L = 16, 2048
_LANES = 16
_CHUNKS = _L // _LANES


def _split_roll_body(ids_hbm, tt_hbm, out1_hbm, out2_hbm,
                     ids_v, tt_v, out1_v, in2_v, out2_v, acc_v, sem1, sem2):
    c = lax.axis_index("c")
    s = lax.axis_index("s")
    wid = s

    if True:
        cpa = pltpu.make_async_copy(ids_hbm.at[wid], ids_v, sem1)
        cpb = pltpu.make_async_copy(tt_hbm.at[wid], tt_v, sem2)
        cpa.start()
        cpb.start()
        cpa.wait()
        cpb.wait()
        acc_v[...] = jnp.zeros((_LANES,), jnp.int32)

        def pass1(j, carry):
            base = j * _LANES
            ids = ids_v[pl.ds(base, _LANES)]
            tt = tt_v[pl.ds(base, _LANES)]
            m0 = jnp.logical_and(tt == 0, ids != 0)
            o1 = jnp.where(m0, ids, 0)
            out1_v[pl.ds(base, _LANES)] = o1
            i2 = ids - o1
            in2_v[pl.ds(base, _LANES)] = i2
            in2_v[pl.ds(base + _L, _LANES)] = i2
            acc_v[...] = acc_v[...] + jnp.where(m0, 1, 0)
            return carry

        lax.fori_loop(0, _CHUNKS, pass1, 0, unroll=2)
        accv = acc_v[...]
        n = accv[0]
        for lane in range(1, _LANES):
            n = n + accv[lane]

        cp1 = pltpu.make_async_copy(out1_v, out1_hbm.at[wid], sem1)
        cp1.start()

        def pass2(j, carry):
            base = j * _LANES
            out2_v[pl.ds(base, _LANES)] = in2_v[pl.ds(base + n, _LANES)]
            return carry

        half = _L // 2
        lax.fori_loop(0, _CHUNKS // 2, pass2, 0, unroll=2)
        cp2a = pltpu.make_async_copy(out2_v.at[pl.ds(0, half)],
                                     out2_hbm.at[wid, pl.ds(0, half)], sem2)
        cp2a.start()
        lax.fori_loop(_CHUNKS // 2, _CHUNKS, pass2, 0, unroll=2)
        cp2b = pltpu.make_async_copy(out2_v.at[pl.ds(half, half)],
                                     out2_hbm.at[wid, pl.ds(half, half)], sem2)
        cp2b.start()
        cp2a.wait()
        cp2b.wait()
        cp1.wait()


def kernel(l_input_ids, token_type_ids):
    mesh = plsc.VectorSubcoreMesh(core_axis_name="c", subcore_axis_name="s",
                                  num_cores=1)
    f = pl.kernel(
        _split_roll_body,
        mesh=mesh,
        out_type=(
            jax.ShapeDtypeStruct((_B, _L), jnp.int32),
            jax.ShapeDtypeStruct((_B, _L), jnp.int32),
        ),
        scratch_types=[
            pltpu.VMEM((_L,), jnp.int32),
            pltpu.VMEM((_L,), jnp.int32),
            pltpu.VMEM((_L,), jnp.int32),
            pltpu.VMEM((2 * _L,), jnp.int32),
            pltpu.VMEM((_L,), jnp.int32),
            pltpu.VMEM((_LANES,), jnp.int32),
            pltpu.SemaphoreType.DMA,
            pltpu.SemaphoreType.DMA,
        ],
    )
    return f(l_input_ids, token_type_ids)
